# baseline (speedup 1.0000x reference)
import functools

import jax
import jax.numpy as jnp
from jax.experimental import pallas as pl

N = 10000
E = 160000


def _rne_bf16(v):
    u = jax.lax.bitcast_convert_type(v, jnp.uint32)
    u = u + jnp.uint32(0x7FFF) + ((u >> 16) & jnp.uint32(1))
    return jax.lax.bitcast_convert_type(
        u & jnp.uint32(0xFFFF0000), jnp.float32)


def _mm(x, w, rne=True):
    n, ci = x.shape
    co = w.shape[1]
    cop = (co + 127) // 128 * 128
    if cop != co:
        w = jnp.pad(w, ((0, 0), (0, cop - co)))
    bn = 1000

    def body(x_ref, w_ref, o_ref):
        if rne:
            a, b = _rne_bf16(x_ref[...]), _rne_bf16(w_ref[...])
        else:
            a, b = x_ref[...], w_ref[...]
        o_ref[...] = jnp.dot(a, b,
                             precision=jax.lax.Precision.HIGHEST,
                             preferred_element_type=jnp.float32)

    out = pl.pallas_call(
        body,
        grid=(n // bn,),
        in_specs=[pl.BlockSpec((bn, ci), lambda i: (i, 0)),
                  pl.BlockSpec((ci, cop), lambda i: (0, 0))],
        out_specs=pl.BlockSpec((bn, cop), lambda i: (i, 0)),
        out_shape=jax.ShapeDtypeStruct((n, cop), jnp.float32),
    )(x, w)
    return out[:, :co] if cop != co else out


def _spline_conv(x, ei, ps, emask, prm, ks, use_mm=True):
    n = x.shape[0]
    src, dst = ei[0], ei[1]
    K = ks * ks
    ci = x.shape[1]
    co = prm['W'].shape[2]
    if use_mm:
        w2 = jnp.concatenate(
            [jnp.transpose(prm['W'], (1, 0, 2)).reshape(ci, K * co),
             prm['root']], axis=1)
        xw_all = _mm(x, w2, rne=(ci > 1))
        xw = xw_all[:, :K * co].reshape(n, K, co)
        xroot = xw_all[:, K * co:]
    else:
        xw = jnp.einsum('ni,kio->kno', x, prm['W'],
                        precision=jax.lax.Precision.HIGHEST).transpose(1, 0, 2)
        xroot = jnp.dot(x, prm['root'],
                        precision=jax.lax.Precision.HIGHEST)

    p = ps * (ks - 1)
    i0 = jnp.clip(jnp.floor(p), 0, ks - 2).astype(jnp.int32)
    f = p - i0.astype(p.dtype)
    msg = jnp.zeros((src.shape[0], co), x.dtype)
    for b0 in (0, 1):
        for b1 in (0, 1):
            w0 = f[:, 0] if b0 else 1.0 - f[:, 0]
            w1 = f[:, 1] if b1 else 1.0 - f[:, 1]
            k = (i0[:, 0] + b0) * ks + (i0[:, 1] + b1)
            msg = msg + (w0 * w1)[:, None] * xw[src, k]
    msg = msg * emask[:, None]
    deg = jnp.clip(jax.ops.segment_sum(emask, dst, num_segments=n), 1.0)
    agg = jax.ops.segment_sum(msg, dst, num_segments=n) / deg[:, None]
    return agg + xroot + prm['bias']


def _cartesian(pos, ei, emask):
    rel = (pos[ei[1]] - pos[ei[0]]) * emask[:, None]
    mx = jnp.maximum(jnp.max(jnp.abs(rel)), 1e-9)
    ps = jnp.clip(rel / (2.0 * mx) + 0.5, 0.0, 1.0)
    return ps * emask[:, None]


def _masked_bn(x, nmask, bn):
    cnt = jnp.clip(jnp.sum(nmask), 1.0)
    mean = jnp.sum(x * nmask[:, None], 0) / cnt
    var = jnp.sum(((x - mean) ** 2) * nmask[:, None], 0) / cnt
    return ((x - mean) / jnp.sqrt(var + 1e-5) * bn['gamma'] + bn['beta']) * nmask[:, None]


def _graclus_pool(x, pos, ei, emask, nmask):
    n = x.shape[0]
    row, col = ei[0], ei[1]
    dist = jnp.sqrt(jnp.sum((pos[row] - pos[col]) ** 2, 1) + 1e-12)
    deg = jnp.clip(jax.ops.segment_sum(emask, row, num_segments=n), 1.0)
    w = dist * (1.0 / deg[row] + 1.0 / deg[col])
    w = jnp.where(emask > 0, w, -jnp.inf)
    best_w = jax.ops.segment_max(w, row, num_segments=n)
    cand = jnp.where((emask > 0) & (w >= best_w[row] - 1e-12), col, n + 1)
    nb = jax.ops.segment_min(cand, row, num_segments=n)
    ar = jnp.arange(n)
    nb = jnp.where(nb >= n, ar, nb)
    mutual = nb[nb] == ar
    cluster = jnp.where(mutual, jnp.minimum(ar, nb), ar)
    is_rep = (cluster == ar)
    rank = jnp.cumsum(is_rep.astype(jnp.int32)) - 1
    inv = rank[cluster]
    u = rank[-1] + 1
    rep = jnp.zeros((n,), jnp.int32).at[jnp.where(is_rep, rank, n)].set(
        ar.astype(jnp.int32), mode='drop')
    live = ar < u
    rep = jnp.where(live, rep, 0)
    partner_raw = nb[rep]
    has2 = mutual[rep] & (partner_raw != rep)
    partner = jnp.where(has2, partner_raw, rep)
    cntf = jnp.where(has2, 2.0, 1.0)
    x_p = jnp.maximum(x[rep], x[partner])
    nmask_p = jnp.maximum(nmask[rep], nmask[partner])
    nmask_p = jnp.where(live, (nmask_p > 0).astype(x.dtype), 0.0)
    x_p = jnp.where(live[:, None], x_p, 0.0) * nmask_p[:, None]
    psum = pos[rep] + jnp.where(has2, 1.0, 0.0)[:, None] * pos[partner]
    pos_p = jnp.where(live[:, None], psum / cntf[:, None], 0.0)
    ei_p = inv[ei]
    emask_p = emask * (ei_p[0] != ei_p[1]).astype(x.dtype)
    ea_p = _cartesian(pos_p, ei_p, emask_p)
    return x_p, pos_p, ei_p, ea_p, emask_p, nmask_p, inv


def kernel(x, edge_index, pos, edge_attr, params):
    n = x.shape[0]
    ei = edge_index
    em = jnp.ones((ei.shape[1],), x.dtype)
    nm = jnp.ones((n,), x.dtype)
    h = jax.nn.elu(_spline_conv(x, ei, edge_attr, em, params['conv1a'], 5,
                                use_mm=True))
    h = jax.nn.elu(_spline_conv(h, ei, edge_attr, em, params['conv1b'], 5,
                                use_mm=True))
    h = _masked_bn(h, nm, params['bn1'])
    h, pos2, ei2, ea2, em2, nm2, inv1 = _graclus_pool(h, pos, ei, em, nm)
    h = jax.nn.elu(_spline_conv(h, ei2, ea2, em2, params['conv2a'], 3,
                                use_mm=True))
    h = jax.nn.elu(_spline_conv(h, ei2, ea2, em2, params['conv2b'], 3,
                                use_mm=True))
    h = _masked_bn(h, nm2, params['bn2'])
    h, pos3, ei3, ea3, em3, nm3, inv2 = _graclus_pool(h, pos2, ei2, em2, nm2)
    h = jax.nn.elu(_spline_conv(h, ei3, ea3, em3, params['conv3a'], 3,
                                use_mm=True))
    h = jax.nn.elu(_spline_conv(h, ei3, ea3, em3, params['conv3b'], 3,
                                use_mm=True))
    h = _masked_bn(h, nm3, params['bn3'])
    h, pos4, ei4, ea4, em4, nm4, inv3 = _graclus_pool(h, pos3, ei3, em3, nm3)
    h = jax.nn.elu(_spline_conv(h, ei4, ea4, em4, params['score_fr'], 5,
                                use_mm=True))
    h = h[inv3]
    h = h[inv2]
    h = h[inv1]
    ea1r = _cartesian(pos, ei, em)
    return _spline_conv(h, ei, ea1r, em, params['convout'], 5, use_mm=True)
